# baseline (speedup 1.0000x reference)
import dataclasses
import functools

import jax
import jax.numpy as jnp
from jax import lax
from jax.experimental import pallas as pl
from jax.experimental.pallas import tpu as pltpu
from jax.experimental.pallas import tpu_sc as plsc

_N = 100000
_D = 128
_DEG = 16
_NB = 4096
_BS = 32

_W = 32
_ABLK = 4096
_NPAD = 25 * _ABLK
_NPW = _NPAD // _W
_CHB = 640
_BPW = _NB // _W
_L = 16


def _compiler_params():
    cp = pltpu.CompilerParams()
    if "needs_layout_passes" in pltpu.CompilerParams.__dataclass_fields__:
        cp = dataclasses.replace(cp, needs_layout_passes=False)
    return cp


def _prep_body(x_ref, t_ref, nbr_ref, h_ref, nbrT_ref):
    h_ref[...] = lax.dot_general(
        x_ref[...], t_ref[...], (((1,), (0,)), ((), ())),
        preferred_element_type=jnp.float32)
    nbrT_ref[...] = jnp.clip(nbr_ref[...].T, 0, _N - 1)


_prep = pl.pallas_call(
    _prep_body,
    grid=(_NPAD // _ABLK,),
    in_specs=[
        pl.BlockSpec((_ABLK, _D), lambda i: (i, 0)),
        pl.BlockSpec((_D, 1), lambda i: (0, 0)),
        pl.BlockSpec((_ABLK, _DEG), lambda i: (i, 0)),
    ],
    out_specs=[
        pl.BlockSpec((_ABLK, 1), lambda i: (i, 0)),
        pl.BlockSpec((_DEG, _ABLK), lambda i: (0, i)),
    ],
    out_shape=[
        jax.ShapeDtypeStruct((_NPAD, 1), jnp.float32),
        jax.ShapeDtypeStruct((_DEG, _NPAD), jnp.int32),
    ],
)


def _make_comb_kernel():
    mesh = plsc.VectorSubcoreMesh(core_axis_name="c", subcore_axis_name="s")

    @functools.partial(
        pl.kernel,
        out_type=jax.ShapeDtypeStruct((_NPAD,), jnp.float32),
        mesh=mesh,
        compiler_params=_compiler_params(),
        scratch_types=[
            pltpu.VMEM((_N,), jnp.float32),
            pltpu.VMEM((_DEG, _CHB), jnp.int32),
            pltpu.VMEM((_CHB,), jnp.float32),
            pltpu.VMEM((_CHB,), jnp.float32),
            pltpu.SemaphoreType.DMA,
        ],
    )
    def comb_kernel(nbrT_hbm, nw_hbm, h_hbm, out_hbm, nw_v, nbr_v, h_v, o_v,
                    sem):
        wid = lax.axis_index("s") * 2 + lax.axis_index("c")
        pltpu.async_copy(nw_hbm, nw_v, sem).wait()
        base0 = wid * _NPW
        for c in range(_NPW // _CHB):
            base = base0 + c * _CHB
            pltpu.sync_copy(nbrT_hbm.at[:, pl.ds(base, _CHB)], nbr_v)
            pltpu.sync_copy(h_hbm.at[pl.ds(base, _CHB)], h_v)

            @pl.loop(0, _CHB // _L)
            def _(i):
                o = i * _L
                acc = plsc.load_gather(nw_v, [nbr_v[0, pl.ds(o, _L)]])
                for d in range(1, _DEG):
                    acc = acc + plsc.load_gather(nw_v,
                                                 [nbr_v[d, pl.ds(o, _L)]])
                o_v[pl.ds(o, _L)] = acc * h_v[pl.ds(o, _L)]

            pltpu.sync_copy(o_v, out_hbm.at[pl.ds(base, _CHB)])

    return comb_kernel


_comb_cache = functools.cache(_make_comb_kernel)


def _make_score_kernel():
    mesh = plsc.VectorSubcoreMesh(core_axis_name="c", subcore_axis_name="s")

    @functools.partial(
        pl.kernel,
        out_type=jax.ShapeDtypeStruct((_NB,), jnp.float32),
        mesh=mesh,
        compiler_params=_compiler_params(),
        scratch_types=[
            pltpu.VMEM((_NPAD,), jnp.float32),
            pltpu.VMEM((_BPW * _BS,), jnp.int32),
            pltpu.VMEM((_BPW * _BS,), jnp.float32),
            pltpu.VMEM((_BPW,), jnp.float32),
            pltpu.SemaphoreType.DMA,
        ],
    )
    def score_kernel(comb_hbm, bags_hbm, alpha_hbm, out_hbm, tab_v, idx_v,
                     a_v, o_v, sem):
        wid = lax.axis_index("s") * 2 + lax.axis_index("c")
        base = wid * _BPW
        pltpu.sync_copy(bags_hbm.at[pl.ds(base * _BS, _BPW * _BS)], idx_v)
        pltpu.sync_copy(alpha_hbm.at[pl.ds(base * _BS, _BPW * _BS)], a_v)
        pltpu.async_copy(comb_hbm, tab_v, sem).wait()

        lane = lax.iota(jnp.int32, _L)

        @pl.loop(0, _BPW // _L)
        def _(g):
            out_vec = jnp.zeros((_L,), jnp.float32)
            for bb in range(_L):
                off = (g * _L + bb) * _BS
                acc = (plsc.load_gather(tab_v, [idx_v[pl.ds(off, _L)]])
                       * a_v[pl.ds(off, _L)])
                for j in range(1, _BS // _L):
                    acc = acc + (
                        plsc.load_gather(tab_v,
                                         [idx_v[pl.ds(off + j * _L, _L)]])
                        * a_v[pl.ds(off + j * _L, _L)])
                out_vec = jnp.where(lane == bb, jnp.sum(acc), out_vec)
            o_v[pl.ds(g * _L, _L)] = out_vec

        pltpu.sync_copy(o_v, out_hbm.at[pl.ds(base, _BPW)])

    return score_kernel


_score_cache = functools.cache(_make_score_kernel)


def kernel(x, sampled_bags, alpha_values, theta_w, node_weights, neighbors):
    h2d, nbrT = _prep(x, theta_w, neighbors)
    comb = _comb_cache()(nbrT, node_weights, h2d.reshape(_NPAD))
    return _score_cache()(comb, sampled_bags.reshape(_NB * _BS),
                          alpha_values.reshape(_NB * _BS))


# speedup vs baseline: 1.6785x; 1.6785x over previous
import dataclasses
import functools

import jax
import jax.numpy as jnp
from jax import lax
from jax.experimental import pallas as pl
from jax.experimental.pallas import tpu as pltpu
from jax.experimental.pallas import tpu_sc as plsc

_N = 100000
_D = 128
_DEG = 16
_NB = 4096
_BS = 32

_W = 32
_ABLK = 4096
_NPAD = 25 * _ABLK
_NPW = _NPAD // _W
_CHB = 640
_BPW = _NB // _W
_L = 16


def _compiler_params():
    cp = pltpu.CompilerParams()
    if "needs_layout_passes" in pltpu.CompilerParams.__dataclass_fields__:
        cp = dataclasses.replace(cp, needs_layout_passes=False)
    return cp


def _proj_body(x_ref, t_ref, h_ref):
    h = lax.dot_general(
        t_ref[...], x_ref[...], (((0,), (1,)), ((), ())),
        preferred_element_type=jnp.float32)
    h_ref[...] = h[0, :]


_proj = pl.pallas_call(
    _proj_body,
    grid=(_NPAD // _ABLK,),
    in_specs=[
        pl.BlockSpec((_ABLK, _D), lambda i: (i, 0)),
        pl.BlockSpec((_D, 1), lambda i: (0, 0)),
    ],
    out_specs=pl.BlockSpec((_ABLK,), lambda i: (i,)),
    out_shape=jax.ShapeDtypeStruct((_NPAD,), jnp.float32),
)


def _make_comb_kernel():
    mesh = plsc.VectorSubcoreMesh(core_axis_name="c", subcore_axis_name="s")

    @functools.partial(
        pl.kernel,
        out_type=jax.ShapeDtypeStruct((_NPAD,), jnp.float32),
        mesh=mesh,
        compiler_params=_compiler_params(),
        scratch_types=[
            pltpu.VMEM((_N,), jnp.float32),
            pltpu.VMEM((_DEG, _CHB), jnp.int32),
            pltpu.VMEM((_CHB,), jnp.float32),
            pltpu.VMEM((_CHB,), jnp.float32),
            pltpu.SemaphoreType.DMA,
        ],
    )
    def comb_kernel(nbrT_hbm, nw_hbm, h_hbm, out_hbm, nw_v, nbr_v, h_v, o_v,
                    sem):
        wid = lax.axis_index("s") * 2 + lax.axis_index("c")
        pltpu.async_copy(nw_hbm, nw_v, sem).wait()
        base0 = wid * _NPW
        for c in range(_NPW // _CHB):
            base = base0 + c * _CHB
            pltpu.sync_copy(nbrT_hbm.at[:, pl.ds(base, _CHB)], nbr_v)
            pltpu.sync_copy(h_hbm.at[pl.ds(base, _CHB)], h_v)

            @pl.loop(0, _CHB // _L)
            def _(i):
                o = i * _L
                acc = plsc.load_gather(nw_v, [nbr_v[0, pl.ds(o, _L)]])
                for d in range(1, _DEG):
                    acc = acc + plsc.load_gather(nw_v,
                                                 [nbr_v[d, pl.ds(o, _L)]])
                o_v[pl.ds(o, _L)] = acc * h_v[pl.ds(o, _L)]

            pltpu.sync_copy(o_v, out_hbm.at[pl.ds(base, _CHB)])

    return comb_kernel


_comb_cache = functools.cache(_make_comb_kernel)


def _make_score_kernel():
    mesh = plsc.VectorSubcoreMesh(core_axis_name="c", subcore_axis_name="s")

    @functools.partial(
        pl.kernel,
        out_type=jax.ShapeDtypeStruct((_NB,), jnp.float32),
        mesh=mesh,
        compiler_params=_compiler_params(),
        scratch_types=[
            pltpu.VMEM((_NPAD,), jnp.float32),
            pltpu.VMEM((_BS, _BPW), jnp.int32),
            pltpu.VMEM((_BS, _BPW), jnp.float32),
            pltpu.VMEM((_BPW,), jnp.float32),
            pltpu.SemaphoreType.DMA,
        ],
    )
    def score_kernel(comb_hbm, bagsT_hbm, alphaT_hbm, out_hbm, tab_v, idx_v,
                     a_v, o_v, sem):
        wid = lax.axis_index("s") * 2 + lax.axis_index("c")
        base = wid * _BPW
        pltpu.sync_copy(bagsT_hbm.at[:, pl.ds(base, _BPW)], idx_v)
        pltpu.sync_copy(alphaT_hbm.at[:, pl.ds(base, _BPW)], a_v)
        pltpu.async_copy(comb_hbm, tab_v, sem).wait()

        @pl.loop(0, _BPW // _L)
        def _(i):
            o = i * _L
            acc = (plsc.load_gather(tab_v, [idx_v[0, pl.ds(o, _L)]])
                   * a_v[0, pl.ds(o, _L)])
            for j in range(1, _BS):
                acc = acc + (plsc.load_gather(tab_v, [idx_v[j, pl.ds(o, _L)]])
                             * a_v[j, pl.ds(o, _L)])
            o_v[pl.ds(o, _L)] = acc

        pltpu.sync_copy(o_v, out_hbm.at[pl.ds(base, _BPW)])

    return score_kernel


_score_cache = functools.cache(_make_score_kernel)


def kernel(x, sampled_bags, alpha_values, theta_w, node_weights, neighbors):
    h = _proj(x, theta_w)
    nbrT = jnp.pad(neighbors.T, ((0, 0), (0, _NPAD - _N)))
    comb = _comb_cache()(nbrT, node_weights, h)
    bagsT = sampled_bags.T
    alphaT = alpha_values[:, :, 0].T
    return _score_cache()(comb, bagsT, alphaT)
